# Initial kernel scaffold; baseline (speedup 1.0000x reference)
import functools

import jax
import jax.numpy as jnp
from jax import lax
from jax.experimental import pallas as pl
from jax.experimental.pallas import tpu as pltpu
from jax.experimental.pallas import tpu_sc as plsc

N = 10000
E = 320000
NFEAT = 128
NHID = 128
NCLASS = 64

NC = 2
NS = 16
NW = NC * NS
CHUNK = 128
K = 79
E_PAD = NW * K * CHUNK
NACC = 10240
ROWS_PER_TILE = NACC // NS
COPIES_PER_TILE = ROWS_PER_TILE // CHUNK


def _make_spmm(D):
  mesh = plsc.VectorSubcoreMesh(core_axis_name="c", subcore_axis_name="s")

  @functools.partial(
      pl.kernel,
      out_type=jax.ShapeDtypeStruct((NC, NACC, D), jnp.float32),
      mesh=mesh,
      scratch_types=[
          pltpu.VMEM((K, CHUNK), jnp.int32),
          pltpu.VMEM((K, CHUNK), jnp.int32),
          pltpu.VMEM((CHUNK, D), jnp.float32),
          pltpu.VMEM_SHARED((NACC, D), jnp.float32),
          pltpu.SemaphoreType.DMA,
      ],
  )
  def spmm(support_hbm, src_hbm, dst_hbm, out_hbm, src_v, dst_v, rows_v,
           acc, sem):
    c = lax.axis_index("c")
    s = lax.axis_index("s")
    wid = s * NC + c

    pltpu.sync_copy(src_hbm.at[wid], src_v)
    pltpu.sync_copy(dst_hbm.at[wid], dst_v)

    zeros16 = jnp.zeros((16,), jnp.float32)

    def zero_row(i, carry):
      for t in range(D // 16):
        rows_v[i, pl.ds(t * 16, 16)] = zeros16
      return carry

    lax.fori_loop(0, CHUNK, zero_row, 0)
    base = s * ROWS_PER_TILE
    for t in range(COPIES_PER_TILE):
      pltpu.sync_copy(rows_v, acc.at[pl.ds(base + t * CHUNK, CHUNK)])
    plsc.subcore_barrier()

    def body(j, carry):
      pltpu.async_copy(support_hbm.at[src_v.at[j]], rows_v, sem).wait()
      pltpu.sync_copy(rows_v, acc.at[dst_v.at[j]], add=True)
      return carry

    lax.fori_loop(0, K, body, 0)
    plsc.subcore_barrier()

    for t in range(COPIES_PER_TILE):
      r0 = base + t * CHUNK
      pltpu.sync_copy(acc.at[pl.ds(r0, CHUNK)], rows_v)
      pltpu.sync_copy(rows_v, out_hbm.at[c, pl.ds(r0, CHUNK)])

  return spmm


_spmm128 = _make_spmm(NHID)
_spmm64 = _make_spmm(NCLASS)

_ROWS_BLK = 1000
_GRID = N // _ROWS_BLK


def _mm_first(x, W):
  def body(x_ref, w_ref, o_ref):
    o_ref[...] = jnp.dot(x_ref[...], w_ref[...],
                         preferred_element_type=jnp.float32)

  return pl.pallas_call(
      body,
      grid=(_GRID,),
      in_specs=[
          pl.BlockSpec((_ROWS_BLK, x.shape[1]), lambda i: (i, 0)),
          pl.BlockSpec(W.shape, lambda i: (0, 0)),
      ],
      out_specs=pl.BlockSpec((_ROWS_BLK, W.shape[1]), lambda i: (i, 0)),
      out_shape=jax.ShapeDtypeStruct((N, W.shape[1]), jnp.float32),
  )(x, W)


def _mm_agg(agg, b, W):
  D = agg.shape[2]

  def body(a_ref, b_ref, w_ref, o_ref):
    h = a_ref[0] + a_ref[1] + b_ref[...]
    o_ref[...] = jnp.dot(h, w_ref[...], preferred_element_type=jnp.float32)

  return pl.pallas_call(
      body,
      grid=(_GRID,),
      in_specs=[
          pl.BlockSpec((NC, _ROWS_BLK, D), lambda i: (0, i, 0)),
          pl.BlockSpec((1, D), lambda i: (0, 0)),
          pl.BlockSpec(W.shape, lambda i: (0, 0)),
      ],
      out_specs=pl.BlockSpec((_ROWS_BLK, W.shape[1]), lambda i: (i, 0)),
      out_shape=jax.ShapeDtypeStruct((N, W.shape[1]), jnp.float32),
  )(agg, b.reshape(1, D), W)


def _softmax_out(agg, b):
  D = agg.shape[2]

  def body(a_ref, b_ref, o_ref):
    z = a_ref[0] + a_ref[1] + b_ref[...]
    z = z - jnp.max(z, axis=1, keepdims=True)
    e = jnp.exp(z)
    o_ref[...] = e / jnp.sum(e, axis=1, keepdims=True)

  return pl.pallas_call(
      body,
      grid=(_GRID,),
      in_specs=[
          pl.BlockSpec((NC, _ROWS_BLK, D), lambda i: (0, i, 0)),
          pl.BlockSpec((1, D), lambda i: (0, 0)),
      ],
      out_specs=pl.BlockSpec((_ROWS_BLK, D), lambda i: (i, 0)),
      out_shape=jax.ShapeDtypeStruct((N, D), jnp.float32),
  )(agg, b.reshape(1, D))


def kernel(x, edge_index, W1, b1, W2, b2, W3, b3):
  src = edge_index[0]
  dst = edge_index[1]
  pad = E_PAD - E
  src_p = jnp.concatenate([src, jnp.zeros((pad,), jnp.int32)])
  dst_p = jnp.concatenate([dst, jnp.full((pad,), N, jnp.int32)])
  src_p = src_p.reshape(NW, K, CHUNK)
  dst_p = dst_p.reshape(NW, K, CHUNK)

  support1 = _mm_first(x, W1)
  agg1 = _spmm128(support1, src_p, dst_p)
  support2 = _mm_agg(agg1, b1, W2)
  agg2 = _spmm128(support2, src_p, dst_p)
  support3 = _mm_agg(agg2, b2, W3)
  agg3 = _spmm64(support3, src_p, dst_p)
  return _softmax_out(agg3, b3)


# speedup vs baseline: 5.0608x; 5.0608x over previous
import functools

import jax
import jax.numpy as jnp
from jax import lax
from jax.experimental import pallas as pl
from jax.experimental.pallas import tpu as pltpu
from jax.experimental.pallas import tpu_sc as plsc

N = 10000
E = 320000
NFEAT = 128
NHID = 128
NCLASS = 64

NC = 2
NS = 16
NW = NC * NS
CHUNK = 128
K = 79
E_PAD = NW * K * CHUNK
NACC = 10240
ROWS_PER_TILE = NACC // NS
COPIES_PER_TILE = ROWS_PER_TILE // CHUNK


def _make_spmm(D):
  mesh = plsc.VectorSubcoreMesh(core_axis_name="c", subcore_axis_name="s")

  @functools.partial(
      pl.kernel,
      out_type=jax.ShapeDtypeStruct((NC, NACC, D), jnp.float32),
      mesh=mesh,
      compiler_params=pltpu.CompilerParams(use_tc_tiling_on_sc=False),
      scratch_types=[
          pltpu.VMEM((K, CHUNK), jnp.int32),
          pltpu.VMEM((K, CHUNK), jnp.int32),
          pltpu.VMEM((CHUNK, D), jnp.float32),
          pltpu.VMEM_SHARED((NACC, D), jnp.float32),
          pltpu.SemaphoreType.DMA,
      ],
  )
  def spmm(support_hbm, src_hbm, dst_hbm, out_hbm, src_v, dst_v, rows_v,
           acc, sem):
    c = lax.axis_index("c")
    s = lax.axis_index("s")
    wid = s * NC + c

    pltpu.sync_copy(src_hbm.at[wid], src_v)
    pltpu.sync_copy(dst_hbm.at[wid], dst_v)

    zeros16 = jnp.zeros((16,), jnp.float32)

    def zero_row(i, carry):
      for t in range(D // 16):
        rows_v[i, pl.ds(t * 16, 16)] = zeros16
      return carry

    lax.fori_loop(0, CHUNK, zero_row, 0)
    base = s * ROWS_PER_TILE
    for t in range(COPIES_PER_TILE):
      pltpu.sync_copy(rows_v, acc.at[pl.ds(base + t * CHUNK, CHUNK)])
    plsc.subcore_barrier()

    def body(j, carry):
      pltpu.async_copy(support_hbm.at[src_v.at[j]], rows_v, sem).wait()
      pltpu.sync_copy(rows_v, acc.at[dst_v.at[j]], add=True)
      return carry

    lax.fori_loop(0, K, body, 0)
    plsc.subcore_barrier()

    for t in range(COPIES_PER_TILE):
      r0 = base + t * CHUNK
      pltpu.sync_copy(acc.at[pl.ds(r0, CHUNK)], rows_v)
      pltpu.sync_copy(rows_v, out_hbm.at[c, pl.ds(r0, CHUNK)])

  return spmm


_spmm128 = _make_spmm(NHID)
_spmm64 = _make_spmm(NCLASS)

_ROWS_BLK = 1000
_GRID = N // _ROWS_BLK


def _mm_first(x, W):
  def body(x_ref, w_ref, o_ref):
    o_ref[...] = jnp.dot(x_ref[...], w_ref[...],
                         preferred_element_type=jnp.float32)

  return pl.pallas_call(
      body,
      grid=(_GRID,),
      in_specs=[
          pl.BlockSpec((_ROWS_BLK, x.shape[1]), lambda i: (i, 0)),
          pl.BlockSpec(W.shape, lambda i: (0, 0)),
      ],
      out_specs=pl.BlockSpec((_ROWS_BLK, W.shape[1]), lambda i: (i, 0)),
      out_shape=jax.ShapeDtypeStruct((N, W.shape[1]), jnp.float32),
  )(x, W)


def _mm_agg(agg, b, W):
  D = agg.shape[2]

  def body(a_ref, b_ref, w_ref, o_ref):
    h = a_ref[0] + a_ref[1] + b_ref[...]
    o_ref[...] = jnp.dot(h, w_ref[...], preferred_element_type=jnp.float32)

  return pl.pallas_call(
      body,
      grid=(_GRID,),
      in_specs=[
          pl.BlockSpec((NC, _ROWS_BLK, D), lambda i: (0, i, 0)),
          pl.BlockSpec((1, D), lambda i: (0, 0)),
          pl.BlockSpec(W.shape, lambda i: (0, 0)),
      ],
      out_specs=pl.BlockSpec((_ROWS_BLK, W.shape[1]), lambda i: (i, 0)),
      out_shape=jax.ShapeDtypeStruct((N, W.shape[1]), jnp.float32),
  )(agg, b.reshape(1, D), W)


def _softmax_out(agg, b):
  D = agg.shape[2]

  def body(a_ref, b_ref, o_ref):
    z = a_ref[0] + a_ref[1] + b_ref[...]
    z = z - jnp.max(z, axis=1, keepdims=True)
    e = jnp.exp(z)
    o_ref[...] = e / jnp.sum(e, axis=1, keepdims=True)

  return pl.pallas_call(
      body,
      grid=(_GRID,),
      in_specs=[
          pl.BlockSpec((NC, _ROWS_BLK, D), lambda i: (0, i, 0)),
          pl.BlockSpec((1, D), lambda i: (0, 0)),
      ],
      out_specs=pl.BlockSpec((_ROWS_BLK, D), lambda i: (i, 0)),
      out_shape=jax.ShapeDtypeStruct((N, D), jnp.float32),
  )(agg, b.reshape(1, D))


def kernel(x, edge_index, W1, b1, W2, b2, W3, b3):
  src = edge_index[0]
  dst = edge_index[1]
  pad = E_PAD - E
  src_p = jnp.concatenate([src, jnp.zeros((pad,), jnp.int32)])
  dst_p = jnp.concatenate([dst, jnp.full((pad,), N, jnp.int32)])
  src_p = src_p.reshape(NW, K, CHUNK)
  dst_p = dst_p.reshape(NW, K, CHUNK)

  support1 = _mm_first(x, W1)
  agg1 = _spmm128(support1, src_p, dst_p)
  support2 = _mm_agg(agg1, b1, W2)
  agg2 = _spmm128(support2, src_p, dst_p)
  support3 = _mm_agg(agg2, b2, W3)
  agg3 = _spmm64(support3, src_p, dst_p)
  return _softmax_out(agg3, b3)
